# baseline (speedup 1.0000x reference)
import functools

import jax
import jax.numpy as jnp
from jax import lax
from jax.experimental import pallas as pl
from jax.experimental.pallas import tpu as pltpu
from jax.experimental.pallas import tpu_sc as plsc

N = 10000
D_IN = 128
D_H = 256
D_OUT = 16
E_PAD = 327680
SUB = 128
NTILES = 16
N_PAD = 10240
RPT = N_PAD // NTILES
TPT = N // NTILES
BLK = 1000
GRID = N // BLK


def _build_agg(dq, nq):
    col_split = nq > 1
    passes = max(1, nq // 2)
    ept = E_PAD // NTILES if col_split else E_PAD // (2 * NTILES)
    ch = 1024
    n_chunks = ept // ch
    nsub = ch // SUB
    nvec = dq // 16
    n_out = nq if col_split else 2
    mesh = plsc.VectorSubcoreMesh(core_axis_name="c", subcore_axis_name="s")

    def body(*refs):
        tabs = refs[:nq]
        src_r, dst_r, ew_f, zeros = refs[nq:nq + 4]
        outs = refs[nq + 4:nq + 4 + n_out]
        (table, acc, src0, src1, dst0, dst1, ewv0, ewv1,
         rb0, rb1, rb2, rb3) = refs[nq + 4 + n_out:nq + 16 + n_out]
        sems = refs[nq + 16 + n_out:]
        gsems = sems[0:4]
        ssems = sems[4:8]
        sts = sems[8:10]

        cid = lax.axis_index("c")
        sid = lax.axis_index("s")
        r0 = sid * RPT
        t0 = sid * TPT
        srcs = (src0, src1)
        dsts = (dst0, dst1)
        ews = (ewv0, ewv1)
        bufs = (rb0, rb1, rb2, rb3)

        if col_split:
            rbase = sid * (ept // SUB)
        else:
            rbase = (cid * NTILES + sid) * (ept // SUB)

        def stage_start(par, ci):
            row0 = rbase + ci * nsub
            pltpu.async_copy(src_r.at[pl.ds(row0, nsub)], srcs[par], sts[par])
            pltpu.async_copy(dst_r.at[pl.ds(row0, nsub)], dsts[par], sts[par])
            pltpu.async_copy(ew_f.at[pl.ds(row0 * SUB, ch)], ews[par], sts[par])

        def stage_wait(par):
            pltpu.make_async_copy(src_r.at[pl.ds(0, nsub)], srcs[par], sts[par]).wait()
            pltpu.make_async_copy(dst_r.at[pl.ds(0, nsub)], dsts[par], sts[par]).wait()
            pltpu.make_async_copy(ew_f.at[pl.ds(0, ch)], ews[par], sts[par]).wait()

        def start_gather(par, s):
            b = s % 4
            pltpu.async_copy(table.at[srcs[par].at[s]], bufs[b], gsems[b])

        def wait_gather(par, s):
            b = s % 4
            pltpu.make_async_copy(
                table.at[srcs[par].at[s]], bufs[b], gsems[b]).wait()

        def start_scatter(par, s):
            b = s % 4
            pltpu.async_copy(bufs[b], acc.at[dsts[par].at[s]], ssems[b],
                             add=True)

        def wait_scatter(par, s):
            b = s % 4
            pltpu.make_async_copy(bufs[b], acc.at[dsts[par].at[s]],
                                  ssems[b]).wait()

        def scale_rows(par, s):
            buf = bufs[s % 4]
            base = jnp.full((16,), s * SUB, jnp.int32)

            @plsc.parallel_loop(0, SUB, unroll=4)
            def _(e):
                wv = plsc.load_gather(ews[par], [base + e])
                for j in range(nvec):
                    sl = pl.ds(j * 16, 16)
                    buf[e, sl] = buf[e, sl] * wv

        def process(par, ci):
            stage_wait(par)
            start_gather(par, 0)
            start_gather(par, 1)
            for s in range(nsub):
                if s >= 2:
                    wait_scatter(par, s - 2)
                if s + 2 < nsub:
                    start_gather(par, s + 2)
                wait_gather(par, s)
                scale_rows(par, s)
                start_scatter(par, s)
            wait_scatter(par, nsub - 2)
            wait_scatter(par, nsub - 1)

            @pl.when(ci + 2 < n_chunks)
            def _():
                stage_start(par, ci + 2)

        def one_pass(p):
            stage_start(0, 0)
            stage_start(1, 1)
            if col_split:
                @pl.when(cid == 0)
                def _():
                    pltpu.sync_copy(tabs[p].at[pl.ds(t0, TPT)],
                                    table.at[pl.ds(t0, TPT)])

                @pl.when(cid == 1)
                def _():
                    pltpu.sync_copy(tabs[passes + p].at[pl.ds(t0, TPT)],
                                    table.at[pl.ds(t0, TPT)])
            else:
                pltpu.sync_copy(tabs[0].at[pl.ds(t0, TPT)],
                                table.at[pl.ds(t0, TPT)])
            pltpu.sync_copy(zeros.at[pl.ds(r0, RPT)], acc.at[pl.ds(r0, RPT)])
            plsc.subcore_barrier()

            def pair(k, carry):
                process(0, 2 * k)
                process(1, 2 * k + 1)
                return carry

            lax.fori_loop(0, n_chunks // 2, pair, 0)
            plsc.subcore_barrier()

            if col_split:
                @pl.when(cid == 0)
                def _():
                    pltpu.sync_copy(acc.at[pl.ds(r0, RPT)],
                                    outs[p].at[pl.ds(r0, RPT)])

                @pl.when(cid == 1)
                def _():
                    pltpu.sync_copy(acc.at[pl.ds(r0, RPT)],
                                    outs[passes + p].at[pl.ds(r0, RPT)])
            else:
                @pl.when(cid == 0)
                def _():
                    pltpu.sync_copy(acc.at[pl.ds(r0, RPT)],
                                    outs[0].at[pl.ds(r0, RPT)])

                @pl.when(cid == 1)
                def _():
                    pltpu.sync_copy(acc.at[pl.ds(r0, RPT)],
                                    outs[1].at[pl.ds(r0, RPT)])

        for p in range(passes):
            one_pass(p)
            if p + 1 < passes:
                plsc.subcore_barrier()

    return pl.kernel(
        body,
        out_type=tuple(
            jax.ShapeDtypeStruct((N_PAD, dq), jnp.float32)
            for _ in range(n_out)),
        mesh=mesh,
        compiler_params=pltpu.CompilerParams(
            needs_layout_passes=False,
            use_tc_tiling_on_sc=False,
        ),
        scratch_types=(
            [pltpu.VMEM_SHARED((N, dq), jnp.float32)]
            + [pltpu.VMEM_SHARED((N_PAD, dq), jnp.float32)]
            + [pltpu.VMEM((nsub, SUB), jnp.int32)] * 2
            + [pltpu.VMEM((nsub, SUB), jnp.int32)] * 2
            + [pltpu.VMEM((ch,), jnp.float32)] * 2
            + [pltpu.VMEM((SUB, dq), jnp.float32)] * 4
            + [pltpu.SemaphoreType.DMA] * 10
        ),
    )


def _blk(shape):
    return pl.BlockSpec(shape, lambda i: (i,) + (0,) * (len(shape) - 1))


def _full(shape):
    return pl.BlockSpec(shape, lambda i: (0,) * len(shape))


def _tc0_body(da, db, xr, dis_ref, u0_ref, u1_ref):
    deg = da[:, 0:1] + db[:, 0:1] + 1.0
    dis = jnp.where(deg > 0, lax.rsqrt(deg), 0.0)
    dis_ref[...] = dis
    u = dis * xr[...]
    u0_ref[...] = u[:, :64]
    u1_ref[...] = u[:, 64:]


def _tc0(deg_a, deg_b, x):
    return pl.pallas_call(
        _tc0_body,
        grid=(GRID,),
        in_specs=[_blk((BLK, D_OUT)), _blk((BLK, D_OUT)), _blk((BLK, D_IN))],
        out_specs=[_blk((BLK, 1)), _blk((BLK, 64)), _blk((BLK, 64))],
        out_shape=[jax.ShapeDtypeStruct((N, 1), jnp.float32),
                   jax.ShapeDtypeStruct((N, 64), jnp.float32),
                   jax.ShapeDtypeStruct((N, 64), jnp.float32)],
    )(deg_a, deg_b, x)


def _tc1_body(s0, s1, xr, dis, w1, b1, w2, *h_refs):
    d = dis[...]
    aggx = d * jnp.concatenate([s0[...], s1[...]], axis=1) + (d * d) * xr[...]
    z1 = jnp.dot(aggx, w1[...], preferred_element_type=jnp.float32) + b1[...]
    h2p = d * jnp.dot(z1, w2[...], preferred_element_type=jnp.float32)
    for q in range(4):
        h_refs[q][...] = h2p[:, 64 * q:64 * (q + 1)]


def _tc1(s10, s11, x, dis, w1, b1, w2):
    return pl.pallas_call(
        _tc1_body,
        grid=(GRID,),
        in_specs=[_blk((BLK, 64)), _blk((BLK, 64)), _blk((BLK, D_IN)),
                  _blk((BLK, 1)), _full((D_IN, D_H)), _full((1, D_H)),
                  _full((D_H, D_H))],
        out_specs=[_blk((BLK, 64))] * 4,
        out_shape=[jax.ShapeDtypeStruct((N, 64), jnp.float32)] * 4,
    )(s10, s11, x, dis, w1, b1, w2)


def _tc2_body(s0, s1, s2, s3, h0, h1, h2, h3, dis, b, w, *o_refs):
    d = dis[...]
    ss = (s0, s1, s2, s3)
    hh = (h0, h1, h2, h3)
    z = jnp.concatenate([d * (ss[q][...] + hh[q][...]) for q in range(4)],
                        axis=1) + b[...]
    hp = d * jnp.dot(z, w[...], preferred_element_type=jnp.float32)
    for q in range(4):
        o_refs[q][...] = hp[:, 64 * q:64 * (q + 1)]


def _tc2(sq, hq, dis, b, w):
    return pl.pallas_call(
        _tc2_body,
        grid=(GRID,),
        in_specs=[_blk((BLK, 64))] * 8
        + [_blk((BLK, 1)), _full((1, D_H)), _full((D_H, D_H))],
        out_specs=[_blk((BLK, 64))] * 4,
        out_shape=[jax.ShapeDtypeStruct((N, 64), jnp.float32)] * 4,
    )(*sq, *hq, dis, b, w)


def _tc3_body(s0, s1, s2, s3, h0, h1, h2, h3, dis, b, w, o_ref):
    d = dis[...]
    ss = (s0, s1, s2, s3)
    hh = (h0, h1, h2, h3)
    z = jnp.concatenate([d * (ss[q][...] + hh[q][...]) for q in range(4)],
                        axis=1) + b[...]
    h = jnp.maximum(z, 0.0)
    o_ref[...] = d * jnp.dot(h, w[...], preferred_element_type=jnp.float32)


def _tc3(sq, hq, dis, b, w):
    return pl.pallas_call(
        _tc3_body,
        grid=(GRID,),
        in_specs=[_blk((BLK, 64))] * 8
        + [_blk((BLK, 1)), _full((1, D_H)), _full((D_H, D_OUT))],
        out_specs=_blk((BLK, D_OUT)),
        out_shape=jax.ShapeDtypeStruct((N, D_OUT), jnp.float32),
    )(*sq, *hq, dis, b, w)


def _tc4_body(sa, sb, hp, dis, b, o_ref):
    d = dis[...]
    o_ref[...] = d * (sa[...] + sb[...] + hp[...]) + b[...]


def _tc4(sa, sb, hp, dis, b):
    return pl.pallas_call(
        _tc4_body,
        grid=(GRID,),
        in_specs=[_blk((BLK, D_OUT)), _blk((BLK, D_OUT)), _blk((BLK, D_OUT)),
                  _blk((BLK, 1)), _full((1, D_OUT))],
        out_specs=_blk((BLK, D_OUT)),
        out_shape=jax.ShapeDtypeStruct((N, D_OUT), jnp.float32),
    )(sa, sb, hp, dis, b)


def kernel(x, edge_index, edge_weight, W1, b1, W2, b2, W3, b3, W4, b4):
    src = edge_index[0]
    dst = edge_index[1]
    e = src.shape[0]
    pad = E_PAD - e
    pad_dst = (jnp.arange(pad, dtype=jnp.int32) * 97) % N
    src_r = jnp.concatenate([src, jnp.zeros((pad,), jnp.int32)]
                            ).reshape(E_PAD // SUB, SUB)
    dst_r = jnp.concatenate([dst, pad_dst]).reshape(E_PAD // SUB, SUB)
    ew_f = jnp.concatenate([edge_weight, jnp.zeros((pad,), jnp.float32)])

    ones16 = jnp.ones((N, D_OUT), jnp.float32)
    zeros16 = jnp.zeros((N_PAD, D_OUT), jnp.float32)
    zeros64 = jnp.zeros((N_PAD, 64), jnp.float32)

    agg16 = _build_agg(D_OUT, 1)
    agg2 = _build_agg(64, 2)
    agg4 = _build_agg(64, 4)

    def trim(arrs):
        return [a[:N] for a in arrs]

    deg_a, deg_b = trim(agg16(ones16, src_r, dst_r, ew_f, zeros16))
    dis, u0, u1 = _tc0(deg_a, deg_b, x)

    s10, s11 = trim(agg2(u0, u1, src_r, dst_r, ew_f, zeros64))
    h2q = _tc1(s10, s11, x, dis, W1, b1.reshape(1, -1), W2)

    s2q = trim(agg4(*h2q, src_r, dst_r, ew_f, zeros64))
    h3q = _tc2(s2q, h2q, dis, b2.reshape(1, -1), W3)

    s3q = trim(agg4(*h3q, src_r, dst_r, ew_f, zeros64))
    h4p = _tc3(s3q, h3q, dis, b3.reshape(1, -1), W4)

    s4a, s4b = trim(agg16(h4p, src_r, dst_r, ew_f, zeros16))
    out = _tc4(s4a, s4b, h4p, dis, b4.reshape(1, -1))
    return (out, 0)


# speedup vs baseline: 1.0583x; 1.0583x over previous
import functools

import jax
import jax.numpy as jnp
from jax import lax
from jax.experimental import pallas as pl
from jax.experimental.pallas import tpu as pltpu
from jax.experimental.pallas import tpu_sc as plsc

N = 10000
D_IN = 128
D_H = 256
D_OUT = 16
E_PAD = 327680
SUB = 64
NTILES = 16
N_PAD = 10240
RPT = N_PAD // NTILES
TPT = N // NTILES
BLK = 1000
GRID = N // BLK


def _build_agg(dq, nq):
    col_split = nq > 1
    passes = max(1, nq // 2)
    ept = E_PAD // NTILES if col_split else E_PAD // (2 * NTILES)
    ch = 1024
    n_chunks = ept // ch
    nsub = ch // SUB
    nvec = dq // 16
    n_out = nq if col_split else 2
    mesh = plsc.VectorSubcoreMesh(core_axis_name="c", subcore_axis_name="s")

    def body(*refs):
        tabs = refs[:nq]
        src_r, dst_r, ew_f, zeros = refs[nq:nq + 4]
        outs = refs[nq + 4:nq + 4 + n_out]
        (table, acc, src0, src1, dst0, dst1, ewv0, ewv1,
         rb0, rb1, rb2, rb3, rb4, rb5) = refs[nq + 4 + n_out:nq + 18 + n_out]
        sems = refs[nq + 18 + n_out:]
        gsems = sems[0:6]
        ssems = sems[6:12]
        sts = sems[12:14]

        cid = lax.axis_index("c")
        sid = lax.axis_index("s")
        r0 = sid * RPT
        t0 = sid * TPT
        srcs = (src0, src1)
        dsts = (dst0, dst1)
        ews = (ewv0, ewv1)
        bufs = (rb0, rb1, rb2, rb3, rb4, rb5)

        if col_split:
            rbase = sid * (ept // SUB)
        else:
            rbase = (cid * NTILES + sid) * (ept // SUB)

        def stage_start(par, ci):
            row0 = rbase + ci * nsub
            pltpu.async_copy(src_r.at[pl.ds(row0, nsub)], srcs[par], sts[par])
            pltpu.async_copy(dst_r.at[pl.ds(row0, nsub)], dsts[par], sts[par])
            pltpu.async_copy(ew_f.at[pl.ds(row0 * SUB, ch)], ews[par], sts[par])

        def stage_wait(par):
            pltpu.make_async_copy(src_r.at[pl.ds(0, nsub)], srcs[par], sts[par]).wait()
            pltpu.make_async_copy(dst_r.at[pl.ds(0, nsub)], dsts[par], sts[par]).wait()
            pltpu.make_async_copy(ew_f.at[pl.ds(0, ch)], ews[par], sts[par]).wait()

        def start_gather(par, s):
            b = s % 6
            pltpu.async_copy(table.at[srcs[par].at[s]], bufs[b], gsems[b])

        def wait_gather(par, s):
            b = s % 6
            pltpu.make_async_copy(
                table.at[srcs[par].at[s]], bufs[b], gsems[b]).wait()

        def start_scatter(par, s):
            b = s % 6
            pltpu.async_copy(bufs[b], acc.at[dsts[par].at[s]], ssems[b],
                             add=True)

        def wait_scatter(par, s):
            b = s % 6
            pltpu.make_async_copy(bufs[b], acc.at[dsts[par].at[s]],
                                  ssems[b]).wait()

        def scale_rows(par, s):
            buf = bufs[s % 6]
            base = jnp.full((16,), s * SUB, jnp.int32)

            @plsc.parallel_loop(0, SUB, unroll=4)
            def _(e):
                wv = plsc.load_gather(ews[par], [base + e])
                for j in range(nvec):
                    sl = pl.ds(j * 16, 16)
                    buf[e, sl] = buf[e, sl] * wv

        def process(par, ci):
            stage_wait(par)
            start_gather(par, 0)
            start_gather(par, 1)
            start_gather(par, 2)
            for s in range(nsub):
                if s >= 3:
                    wait_scatter(par, s - 3)
                if s + 3 < nsub:
                    start_gather(par, s + 3)
                wait_gather(par, s)
                scale_rows(par, s)
                start_scatter(par, s)
            wait_scatter(par, nsub - 3)
            wait_scatter(par, nsub - 2)
            wait_scatter(par, nsub - 1)

            @pl.when(ci + 2 < n_chunks)
            def _():
                stage_start(par, ci + 2)

        def one_pass(p):
            stage_start(0, 0)
            stage_start(1, 1)
            if col_split:
                @pl.when(cid == 0)
                def _():
                    pltpu.sync_copy(tabs[p].at[pl.ds(t0, TPT)],
                                    table.at[pl.ds(t0, TPT)])

                @pl.when(cid == 1)
                def _():
                    pltpu.sync_copy(tabs[passes + p].at[pl.ds(t0, TPT)],
                                    table.at[pl.ds(t0, TPT)])
            else:
                pltpu.sync_copy(tabs[0].at[pl.ds(t0, TPT)],
                                table.at[pl.ds(t0, TPT)])
            pltpu.sync_copy(zeros.at[pl.ds(r0, RPT)], acc.at[pl.ds(r0, RPT)])
            plsc.subcore_barrier()

            def pair(k, carry):
                process(0, 2 * k)
                process(1, 2 * k + 1)
                return carry

            lax.fori_loop(0, n_chunks // 2, pair, 0)
            plsc.subcore_barrier()

            if col_split:
                @pl.when(cid == 0)
                def _():
                    pltpu.sync_copy(acc.at[pl.ds(r0, RPT)],
                                    outs[p].at[pl.ds(r0, RPT)])

                @pl.when(cid == 1)
                def _():
                    pltpu.sync_copy(acc.at[pl.ds(r0, RPT)],
                                    outs[passes + p].at[pl.ds(r0, RPT)])
            else:
                @pl.when(cid == 0)
                def _():
                    pltpu.sync_copy(acc.at[pl.ds(r0, RPT)],
                                    outs[0].at[pl.ds(r0, RPT)])

                @pl.when(cid == 1)
                def _():
                    pltpu.sync_copy(acc.at[pl.ds(r0, RPT)],
                                    outs[1].at[pl.ds(r0, RPT)])

        for p in range(passes):
            one_pass(p)
            if p + 1 < passes:
                plsc.subcore_barrier()

    return pl.kernel(
        body,
        out_type=tuple(
            jax.ShapeDtypeStruct((N_PAD, dq), jnp.float32)
            for _ in range(n_out)),
        mesh=mesh,
        compiler_params=pltpu.CompilerParams(
            needs_layout_passes=False,
            use_tc_tiling_on_sc=False,
        ),
        scratch_types=(
            [pltpu.VMEM_SHARED((N, dq), jnp.float32)]
            + [pltpu.VMEM_SHARED((N_PAD, dq), jnp.float32)]
            + [pltpu.VMEM((nsub, SUB), jnp.int32)] * 2
            + [pltpu.VMEM((nsub, SUB), jnp.int32)] * 2
            + [pltpu.VMEM((ch,), jnp.float32)] * 2
            + [pltpu.VMEM((SUB, dq), jnp.float32)] * 6
            + [pltpu.SemaphoreType.DMA] * 14
        ),
    )


def _blk(shape):
    return pl.BlockSpec(shape, lambda i: (i,) + (0,) * (len(shape) - 1))


def _full(shape):
    return pl.BlockSpec(shape, lambda i: (0,) * len(shape))


def _tc0_body(da, db, xr, dis_ref, u0_ref, u1_ref):
    deg = da[:, 0:1] + db[:, 0:1] + 1.0
    dis = jnp.where(deg > 0, lax.rsqrt(deg), 0.0)
    dis_ref[...] = dis
    u = dis * xr[...]
    u0_ref[...] = u[:, :64]
    u1_ref[...] = u[:, 64:]


def _tc0(deg_a, deg_b, x):
    return pl.pallas_call(
        _tc0_body,
        grid=(GRID,),
        in_specs=[_blk((BLK, D_OUT)), _blk((BLK, D_OUT)), _blk((BLK, D_IN))],
        out_specs=[_blk((BLK, 1)), _blk((BLK, 64)), _blk((BLK, 64))],
        out_shape=[jax.ShapeDtypeStruct((N, 1), jnp.float32),
                   jax.ShapeDtypeStruct((N, 64), jnp.float32),
                   jax.ShapeDtypeStruct((N, 64), jnp.float32)],
    )(deg_a, deg_b, x)


def _tc1_body(s0, s1, xr, dis, w1, b1, w2, *h_refs):
    d = dis[...]
    aggx = d * jnp.concatenate([s0[...], s1[...]], axis=1) + (d * d) * xr[...]
    z1 = jnp.dot(aggx, w1[...], preferred_element_type=jnp.float32) + b1[...]
    h2p = d * jnp.dot(z1, w2[...], preferred_element_type=jnp.float32)
    for q in range(4):
        h_refs[q][...] = h2p[:, 64 * q:64 * (q + 1)]


def _tc1(s10, s11, x, dis, w1, b1, w2):
    return pl.pallas_call(
        _tc1_body,
        grid=(GRID,),
        in_specs=[_blk((BLK, 64)), _blk((BLK, 64)), _blk((BLK, D_IN)),
                  _blk((BLK, 1)), _full((D_IN, D_H)), _full((1, D_H)),
                  _full((D_H, D_H))],
        out_specs=[_blk((BLK, 64))] * 4,
        out_shape=[jax.ShapeDtypeStruct((N, 64), jnp.float32)] * 4,
    )(s10, s11, x, dis, w1, b1, w2)


def _tc2_body(s0, s1, s2, s3, h0, h1, h2, h3, dis, b, w, *o_refs):
    d = dis[...]
    ss = (s0, s1, s2, s3)
    hh = (h0, h1, h2, h3)
    z = jnp.concatenate([d * (ss[q][...] + hh[q][...]) for q in range(4)],
                        axis=1) + b[...]
    hp = d * jnp.dot(z, w[...], preferred_element_type=jnp.float32)
    for q in range(4):
        o_refs[q][...] = hp[:, 64 * q:64 * (q + 1)]


def _tc2(sq, hq, dis, b, w):
    return pl.pallas_call(
        _tc2_body,
        grid=(GRID,),
        in_specs=[_blk((BLK, 64))] * 8
        + [_blk((BLK, 1)), _full((1, D_H)), _full((D_H, D_H))],
        out_specs=[_blk((BLK, 64))] * 4,
        out_shape=[jax.ShapeDtypeStruct((N, 64), jnp.float32)] * 4,
    )(*sq, *hq, dis, b, w)


def _tc3_body(s0, s1, s2, s3, h0, h1, h2, h3, dis, b, w, o_ref):
    d = dis[...]
    ss = (s0, s1, s2, s3)
    hh = (h0, h1, h2, h3)
    z = jnp.concatenate([d * (ss[q][...] + hh[q][...]) for q in range(4)],
                        axis=1) + b[...]
    h = jnp.maximum(z, 0.0)
    o_ref[...] = d * jnp.dot(h, w[...], preferred_element_type=jnp.float32)


def _tc3(sq, hq, dis, b, w):
    return pl.pallas_call(
        _tc3_body,
        grid=(GRID,),
        in_specs=[_blk((BLK, 64))] * 8
        + [_blk((BLK, 1)), _full((1, D_H)), _full((D_H, D_OUT))],
        out_specs=_blk((BLK, D_OUT)),
        out_shape=jax.ShapeDtypeStruct((N, D_OUT), jnp.float32),
    )(*sq, *hq, dis, b, w)


def _tc4_body(sa, sb, hp, dis, b, o_ref):
    d = dis[...]
    o_ref[...] = d * (sa[...] + sb[...] + hp[...]) + b[...]


def _tc4(sa, sb, hp, dis, b):
    return pl.pallas_call(
        _tc4_body,
        grid=(GRID,),
        in_specs=[_blk((BLK, D_OUT)), _blk((BLK, D_OUT)), _blk((BLK, D_OUT)),
                  _blk((BLK, 1)), _full((1, D_OUT))],
        out_specs=_blk((BLK, D_OUT)),
        out_shape=jax.ShapeDtypeStruct((N, D_OUT), jnp.float32),
    )(sa, sb, hp, dis, b)


def kernel(x, edge_index, edge_weight, W1, b1, W2, b2, W3, b3, W4, b4):
    src = edge_index[0]
    dst = edge_index[1]
    e = src.shape[0]
    pad = E_PAD - e
    pad_dst = (jnp.arange(pad, dtype=jnp.int32) * 97) % N
    src_r = jnp.concatenate([src, jnp.zeros((pad,), jnp.int32)]
                            ).reshape(E_PAD // SUB, SUB)
    dst_r = jnp.concatenate([dst, pad_dst]).reshape(E_PAD // SUB, SUB)
    ew_f = jnp.concatenate([edge_weight, jnp.zeros((pad,), jnp.float32)])

    ones16 = jnp.ones((N, D_OUT), jnp.float32)
    zeros16 = jnp.zeros((N_PAD, D_OUT), jnp.float32)
    zeros64 = jnp.zeros((N_PAD, 64), jnp.float32)

    agg16 = _build_agg(D_OUT, 1)
    agg2 = _build_agg(64, 2)
    agg4 = _build_agg(64, 4)

    def trim(arrs):
        return [a[:N] for a in arrs]

    deg_a, deg_b = trim(agg16(ones16, src_r, dst_r, ew_f, zeros16))
    dis, u0, u1 = _tc0(deg_a, deg_b, x)

    s10, s11 = trim(agg2(u0, u1, src_r, dst_r, ew_f, zeros64))
    h2q = _tc1(s10, s11, x, dis, W1, b1.reshape(1, -1), W2)

    s2q = trim(agg4(*h2q, src_r, dst_r, ew_f, zeros64))
    h3q = _tc2(s2q, h2q, dis, b2.reshape(1, -1), W3)

    s3q = trim(agg4(*h3q, src_r, dst_r, ew_f, zeros64))
    h4p = _tc3(s3q, h3q, dis, b3.reshape(1, -1), W4)

    s4a, s4b = trim(agg16(h4p, src_r, dst_r, ew_f, zeros16))
    out = _tc4(s4a, s4b, h4p, dis, b4.reshape(1, -1))
    return (out, 0)


# speedup vs baseline: 1.1009x; 1.0402x over previous
import functools

import jax
import jax.numpy as jnp
from jax import lax
from jax.experimental import pallas as pl
from jax.experimental.pallas import tpu as pltpu
from jax.experimental.pallas import tpu_sc as plsc

N = 10000
D_IN = 128
D_H = 256
D_OUT = 16
E_PAD = 327680
SUB = 64
NTILES = 16
N_PAD = 10240
RPT = N_PAD // NTILES
TPT = N // NTILES
BLK = 1000
GRID = N // BLK


def _build_agg(dq, nq):
    col_split = nq > 1
    passes = max(1, nq // 2)
    ept = E_PAD // NTILES if col_split else E_PAD // (2 * NTILES)
    ch = 1024
    n_chunks = ept // ch
    nsub = ch // SUB
    nvec = dq // 16
    n_out = nq if col_split else 2
    mesh = plsc.VectorSubcoreMesh(core_axis_name="c", subcore_axis_name="s")

    def body(*refs):
        tabs = refs[:nq]
        src_r, dst_r, ew_f, zeros = refs[nq:nq + 4]
        outs = refs[nq + 4:nq + 4 + n_out]
        (table, acc, src0, src1, dst0, dst1, ewv0, ewv1,
         rb0, rb1, rb2, rb3) = refs[nq + 4 + n_out:nq + 16 + n_out]
        sems = refs[nq + 16 + n_out:]
        gsems = sems[0:4]
        ssems = sems[4:8]
        sts = sems[8:10]

        cid = lax.axis_index("c")
        sid = lax.axis_index("s")
        r0 = sid * RPT
        t0 = sid * TPT
        srcs = (src0, src1)
        dsts = (dst0, dst1)
        ews = (ewv0, ewv1)
        bufs = (rb0, rb1, rb2, rb3)

        if col_split:
            rbase = sid * (ept // SUB)
        else:
            rbase = (cid * NTILES + sid) * (ept // SUB)

        def stage_start(par, ci):
            row0 = rbase + ci * nsub
            pltpu.async_copy(src_r.at[pl.ds(row0, nsub)], srcs[par], sts[par])
            pltpu.async_copy(dst_r.at[pl.ds(row0, nsub)], dsts[par], sts[par])
            pltpu.async_copy(ew_f.at[pl.ds(row0 * SUB, ch)], ews[par], sts[par])

        def stage_wait(par):
            pltpu.make_async_copy(src_r.at[pl.ds(0, nsub)], srcs[par], sts[par]).wait()
            pltpu.make_async_copy(dst_r.at[pl.ds(0, nsub)], dsts[par], sts[par]).wait()
            pltpu.make_async_copy(ew_f.at[pl.ds(0, ch)], ews[par], sts[par]).wait()

        def start_gather(par, s):
            b = s % 4
            pltpu.async_copy(table.at[srcs[par].at[s]], bufs[b], gsems[b])

        def wait_gather(par, s):
            b = s % 4
            pltpu.make_async_copy(
                table.at[srcs[par].at[s]], bufs[b], gsems[b]).wait()

        def start_scatter(par, s):
            b = s % 4
            pltpu.async_copy(bufs[b], acc.at[dsts[par].at[s]], ssems[b],
                             add=True)

        def wait_scatter(par, s):
            b = s % 4
            pltpu.make_async_copy(bufs[b], acc.at[dsts[par].at[s]],
                                  ssems[b]).wait()

        def scale_rows(par, s):
            buf = bufs[s % 4]
            base = jnp.full((16,), s * SUB, jnp.int32)

            @plsc.parallel_loop(0, SUB, unroll=4)
            def _(e):
                wv = plsc.load_gather(ews[par], [base + e])
                for j in range(nvec):
                    sl = pl.ds(j * 16, 16)
                    buf[e, sl] = buf[e, sl] * wv

        def process(par, ci):
            stage_wait(par)
            start_gather(par, 0)
            start_gather(par, 1)
            for s in range(nsub):
                if s >= 2:
                    wait_scatter(par, s - 2)
                if s + 2 < nsub:
                    start_gather(par, s + 2)
                wait_gather(par, s)
                scale_rows(par, s)
                start_scatter(par, s)
            wait_scatter(par, nsub - 2)
            wait_scatter(par, nsub - 1)

            @pl.when(ci + 2 < n_chunks)
            def _():
                stage_start(par, ci + 2)

        def one_pass(p):
            stage_start(0, 0)
            stage_start(1, 1)
            if col_split:
                @pl.when(cid == 0)
                def _():
                    pltpu.sync_copy(tabs[p].at[pl.ds(t0, TPT)],
                                    table.at[pl.ds(t0, TPT)])

                @pl.when(cid == 1)
                def _():
                    pltpu.sync_copy(tabs[passes + p].at[pl.ds(t0, TPT)],
                                    table.at[pl.ds(t0, TPT)])
            else:
                pltpu.sync_copy(tabs[0].at[pl.ds(t0, TPT)],
                                table.at[pl.ds(t0, TPT)])
            pltpu.sync_copy(zeros.at[pl.ds(r0, RPT)], acc.at[pl.ds(r0, RPT)])
            plsc.subcore_barrier()

            def pair(k, carry):
                process(0, 2 * k)
                process(1, 2 * k + 1)
                return carry

            lax.fori_loop(0, n_chunks // 2, pair, 0)
            plsc.subcore_barrier()

            if col_split:
                @pl.when(cid == 0)
                def _():
                    pltpu.sync_copy(acc.at[pl.ds(r0, RPT)],
                                    outs[p].at[pl.ds(r0, RPT)])

                @pl.when(cid == 1)
                def _():
                    pltpu.sync_copy(acc.at[pl.ds(r0, RPT)],
                                    outs[passes + p].at[pl.ds(r0, RPT)])
            else:
                @pl.when(cid == 0)
                def _():
                    pltpu.sync_copy(acc.at[pl.ds(r0, RPT)],
                                    outs[0].at[pl.ds(r0, RPT)])

                @pl.when(cid == 1)
                def _():
                    pltpu.sync_copy(acc.at[pl.ds(r0, RPT)],
                                    outs[1].at[pl.ds(r0, RPT)])

        for p in range(passes):
            one_pass(p)
            if p + 1 < passes:
                plsc.subcore_barrier()

    return pl.kernel(
        body,
        out_type=tuple(
            jax.ShapeDtypeStruct((N_PAD, dq), jnp.float32)
            for _ in range(n_out)),
        mesh=mesh,
        compiler_params=pltpu.CompilerParams(
            needs_layout_passes=False,
            use_tc_tiling_on_sc=False,
        ),
        scratch_types=(
            [pltpu.VMEM_SHARED((N, dq), jnp.float32)]
            + [pltpu.VMEM_SHARED((N_PAD, dq), jnp.float32)]
            + [pltpu.VMEM((nsub, SUB), jnp.int32)] * 2
            + [pltpu.VMEM((nsub, SUB), jnp.int32)] * 2
            + [pltpu.VMEM((ch,), jnp.float32)] * 2
            + [pltpu.VMEM((SUB, dq), jnp.float32)] * 4
            + [pltpu.SemaphoreType.DMA] * 10
        ),
    )


def _blk(shape):
    return pl.BlockSpec(shape, lambda i: (i,) + (0,) * (len(shape) - 1))


def _full(shape):
    return pl.BlockSpec(shape, lambda i: (0,) * len(shape))


def _tc0_body(da, db, xr, dis_ref, u0_ref, u1_ref):
    deg = da[:, 0:1] + db[:, 0:1] + 1.0
    dis = jnp.where(deg > 0, lax.rsqrt(deg), 0.0)
    dis_ref[...] = dis
    u = dis * xr[...]
    u0_ref[...] = u[:, :64]
    u1_ref[...] = u[:, 64:]


def _tc0(deg_a, deg_b, x):
    return pl.pallas_call(
        _tc0_body,
        grid=(GRID,),
        in_specs=[_blk((BLK, D_OUT)), _blk((BLK, D_OUT)), _blk((BLK, D_IN))],
        out_specs=[_blk((BLK, 1)), _blk((BLK, 64)), _blk((BLK, 64))],
        out_shape=[jax.ShapeDtypeStruct((N, 1), jnp.float32),
                   jax.ShapeDtypeStruct((N, 64), jnp.float32),
                   jax.ShapeDtypeStruct((N, 64), jnp.float32)],
    )(deg_a, deg_b, x)


def _tc1_body(s0, s1, xr, dis, w1, b1, w2, *h_refs):
    d = dis[...]
    aggx = d * jnp.concatenate([s0[...], s1[...]], axis=1) + (d * d) * xr[...]
    z1 = jnp.dot(aggx, w1[...], preferred_element_type=jnp.float32) + b1[...]
    h2p = d * jnp.dot(z1, w2[...], preferred_element_type=jnp.float32)
    for q in range(4):
        h_refs[q][...] = h2p[:, 64 * q:64 * (q + 1)]


def _tc1(s10, s11, x, dis, w1, b1, w2):
    return pl.pallas_call(
        _tc1_body,
        grid=(GRID,),
        in_specs=[_blk((BLK, 64)), _blk((BLK, 64)), _blk((BLK, D_IN)),
                  _blk((BLK, 1)), _full((D_IN, D_H)), _full((1, D_H)),
                  _full((D_H, D_H))],
        out_specs=[_blk((BLK, 64))] * 4,
        out_shape=[jax.ShapeDtypeStruct((N, 64), jnp.float32)] * 4,
    )(s10, s11, x, dis, w1, b1, w2)


def _tc2_body(s0, s1, s2, s3, h0, h1, h2, h3, dis, b, w, *o_refs):
    d = dis[...]
    ss = (s0, s1, s2, s3)
    hh = (h0, h1, h2, h3)
    z = jnp.concatenate([d * (ss[q][...] + hh[q][...]) for q in range(4)],
                        axis=1) + b[...]
    hp = d * jnp.dot(z, w[...], preferred_element_type=jnp.float32)
    for q in range(4):
        o_refs[q][...] = hp[:, 64 * q:64 * (q + 1)]


def _tc2(sq, hq, dis, b, w):
    return pl.pallas_call(
        _tc2_body,
        grid=(GRID,),
        in_specs=[_blk((BLK, 64))] * 8
        + [_blk((BLK, 1)), _full((1, D_H)), _full((D_H, D_H))],
        out_specs=[_blk((BLK, 64))] * 4,
        out_shape=[jax.ShapeDtypeStruct((N, 64), jnp.float32)] * 4,
    )(*sq, *hq, dis, b, w)


def _tc3_body(s0, s1, s2, s3, h0, h1, h2, h3, dis, b, w, o_ref):
    d = dis[...]
    ss = (s0, s1, s2, s3)
    hh = (h0, h1, h2, h3)
    z = jnp.concatenate([d * (ss[q][...] + hh[q][...]) for q in range(4)],
                        axis=1) + b[...]
    h = jnp.maximum(z, 0.0)
    o_ref[...] = d * jnp.dot(h, w[...], preferred_element_type=jnp.float32)


def _tc3(sq, hq, dis, b, w):
    return pl.pallas_call(
        _tc3_body,
        grid=(GRID,),
        in_specs=[_blk((BLK, 64))] * 8
        + [_blk((BLK, 1)), _full((1, D_H)), _full((D_H, D_OUT))],
        out_specs=_blk((BLK, D_OUT)),
        out_shape=jax.ShapeDtypeStruct((N, D_OUT), jnp.float32),
    )(*sq, *hq, dis, b, w)


def _tc4_body(sa, sb, hp, dis, b, o_ref):
    d = dis[...]
    o_ref[...] = d * (sa[...] + sb[...] + hp[...]) + b[...]


def _tc4(sa, sb, hp, dis, b):
    return pl.pallas_call(
        _tc4_body,
        grid=(GRID,),
        in_specs=[_blk((BLK, D_OUT)), _blk((BLK, D_OUT)), _blk((BLK, D_OUT)),
                  _blk((BLK, 1)), _full((1, D_OUT))],
        out_specs=_blk((BLK, D_OUT)),
        out_shape=jax.ShapeDtypeStruct((N, D_OUT), jnp.float32),
    )(sa, sb, hp, dis, b)


def kernel(x, edge_index, edge_weight, W1, b1, W2, b2, W3, b3, W4, b4):
    src = edge_index[0]
    dst = edge_index[1]
    e = src.shape[0]
    pad = E_PAD - e
    pad_dst = (jnp.arange(pad, dtype=jnp.int32) * 97) % N
    src_r = jnp.concatenate([src, jnp.zeros((pad,), jnp.int32)]
                            ).reshape(E_PAD // SUB, SUB)
    dst_r = jnp.concatenate([dst, pad_dst]).reshape(E_PAD // SUB, SUB)
    ew_f = jnp.concatenate([edge_weight, jnp.zeros((pad,), jnp.float32)])

    ones16 = jnp.ones((N, D_OUT), jnp.float32)
    zeros16 = jnp.zeros((N_PAD, D_OUT), jnp.float32)
    zeros64 = jnp.zeros((N_PAD, 64), jnp.float32)

    agg16 = _build_agg(D_OUT, 1)
    agg2 = _build_agg(64, 2)
    agg4 = _build_agg(64, 4)

    def trim(arrs):
        return [a[:N] for a in arrs]

    deg_a, deg_b = trim(agg16(ones16, src_r, dst_r, ew_f, zeros16))
    dis, u0, u1 = _tc0(deg_a, deg_b, x)

    s10, s11 = trim(agg2(u0, u1, src_r, dst_r, ew_f, zeros64))
    h2q = _tc1(s10, s11, x, dis, W1, b1.reshape(1, -1), W2)

    s2q = trim(agg4(*h2q, src_r, dst_r, ew_f, zeros64))
    h3q = _tc2(s2q, h2q, dis, b2.reshape(1, -1), W3)

    s3q = trim(agg4(*h3q, src_r, dst_r, ew_f, zeros64))
    h4p = _tc3(s3q, h3q, dis, b3.reshape(1, -1), W4)

    s4a, s4b = trim(agg16(h4p, src_r, dst_r, ew_f, zeros16))
    out = _tc4(s4a, s4b, h4p, dis, b4.reshape(1, -1))
    return (out, 0)


# speedup vs baseline: 1.1011x; 1.0002x over previous
import jax
import jax.numpy as jnp
from jax import lax
from jax.experimental import pallas as pl
from jax.experimental.pallas import tpu as pltpu
from jax.experimental.pallas import tpu_sc as plsc

N = 10000
D_IN = 128
D_H = 256
D_OUT = 16
E_PAD = 327680
SUB = 64
NTILES = 16
N_PAD = 10240
RPT = N_PAD // NTILES
TPT = N // NTILES
BLK = 1000
GRID = N // BLK


def _build_agg(dq, nq):
    col_split = nq > 1
    passes = max(1, nq // 2)
    ept = E_PAD // NTILES if col_split else E_PAD // (2 * NTILES)
    ch = 1024
    n_chunks = ept // ch
    nsub = ch // SUB
    nvec = dq // 16
    n_out = nq if col_split else 2
    mesh = plsc.VectorSubcoreMesh(core_axis_name="c", subcore_axis_name="s")

    def body(*refs):
        tabs = refs[:nq]
        src_r, dst_r, ew_f, zeros = refs[nq:nq + 4]
        outs = refs[nq + 4:nq + 4 + n_out]
        (table, acc, src0, src1, dst0, dst1, ewv0, ewv1,
         rb0, rb1, rb2, rb3) = refs[nq + 4 + n_out:nq + 16 + n_out]
        sems = refs[nq + 16 + n_out:]
        gsems = sems[0:4]
        ssems = sems[4:8]
        sts = sems[8:10]

        cid = lax.axis_index("c")
        sid = lax.axis_index("s")
        r0 = sid * RPT
        t0 = sid * TPT
        srcs = (src0, src1)
        dsts = (dst0, dst1)
        ews = (ewv0, ewv1)
        bufs = (rb0, rb1, rb2, rb3)

        if col_split:
            rbase = sid * (ept // SUB)
        else:
            rbase = (cid * NTILES + sid) * (ept // SUB)

        def stage_start(par, ci):
            row0 = rbase + ci * nsub
            pltpu.async_copy(src_r.at[pl.ds(row0, nsub)], srcs[par], sts[par])
            pltpu.async_copy(dst_r.at[pl.ds(row0, nsub)], dsts[par], sts[par])
            pltpu.async_copy(ew_f.at[pl.ds(row0 * SUB, ch)], ews[par], sts[par])

        def stage_wait(par):
            pltpu.make_async_copy(src_r.at[pl.ds(0, nsub)], srcs[par], sts[par]).wait()
            pltpu.make_async_copy(dst_r.at[pl.ds(0, nsub)], dsts[par], sts[par]).wait()
            pltpu.make_async_copy(ew_f.at[pl.ds(0, ch)], ews[par], sts[par]).wait()

        def start_gather(par, s):
            b = s % 4
            pltpu.async_copy(table.at[srcs[par].at[s]], bufs[b], gsems[b])

        def wait_gather(par, s):
            b = s % 4
            pltpu.make_async_copy(
                table.at[srcs[par].at[s]], bufs[b], gsems[b]).wait()

        def start_scatter(par, s):
            b = s % 4
            pltpu.async_copy(bufs[b], acc.at[dsts[par].at[s]], ssems[b],
                             add=True)

        def wait_scatter(par, s):
            b = s % 4
            pltpu.make_async_copy(bufs[b], acc.at[dsts[par].at[s]],
                                  ssems[b]).wait()

        def scale_rows(par, s):
            buf = bufs[s % 4]
            base = jnp.full((16,), s * SUB, jnp.int32)

            @plsc.parallel_loop(0, SUB, unroll=4)
            def _(e):
                wv = plsc.load_gather(ews[par], [base + e])
                for j in range(nvec):
                    sl = pl.ds(j * 16, 16)
                    buf[e, sl] = buf[e, sl] * wv

        def process(par, ci):
            stage_wait(par)
            start_gather(par, 0)
            start_gather(par, 1)
            for s in range(nsub):
                if s >= 2:
                    wait_scatter(par, s - 2)
                if s + 2 < nsub:
                    start_gather(par, s + 2)
                wait_gather(par, s)
                scale_rows(par, s)
                start_scatter(par, s)
            wait_scatter(par, nsub - 2)
            wait_scatter(par, nsub - 1)

            @pl.when(ci + 2 < n_chunks)
            def _():
                stage_start(par, ci + 2)

        def one_pass(p):
            stage_start(0, 0)
            stage_start(1, 1)
            if col_split:
                @pl.when(cid == 0)
                def _():
                    pltpu.sync_copy(tabs[p].at[pl.ds(t0, TPT)],
                                    table.at[pl.ds(t0, TPT)])

                @pl.when(cid == 1)
                def _():
                    pltpu.sync_copy(tabs[passes + p].at[pl.ds(t0, TPT)],
                                    table.at[pl.ds(t0, TPT)])
            else:
                pltpu.sync_copy(tabs[0].at[pl.ds(t0, TPT)],
                                table.at[pl.ds(t0, TPT)])
            pltpu.sync_copy(zeros.at[pl.ds(r0, RPT)], acc.at[pl.ds(r0, RPT)])
            plsc.subcore_barrier()

            def pair(k, carry):
                process(0, 2 * k)
                process(1, 2 * k + 1)
                return carry

            lax.fori_loop(0, n_chunks // 2, pair, 0)
            plsc.subcore_barrier()

            if col_split:
                @pl.when(cid == 0)
                def _():
                    pltpu.sync_copy(acc.at[pl.ds(r0, RPT)],
                                    outs[p].at[pl.ds(r0, RPT)])

                @pl.when(cid == 1)
                def _():
                    pltpu.sync_copy(acc.at[pl.ds(r0, RPT)],
                                    outs[passes + p].at[pl.ds(r0, RPT)])
            else:
                @pl.when(cid == 0)
                def _():
                    pltpu.sync_copy(acc.at[pl.ds(r0, RPT)],
                                    outs[0].at[pl.ds(r0, RPT)])

                @pl.when(cid == 1)
                def _():
                    pltpu.sync_copy(acc.at[pl.ds(r0, RPT)],
                                    outs[1].at[pl.ds(r0, RPT)])

        for p in range(passes):
            one_pass(p)
            if p + 1 < passes:
                plsc.subcore_barrier()

    return pl.kernel(
        body,
        out_type=tuple(
            jax.ShapeDtypeStruct((N_PAD, dq), jnp.float32)
            for _ in range(n_out)),
        mesh=mesh,
        compiler_params=pltpu.CompilerParams(
            needs_layout_passes=False,
            use_tc_tiling_on_sc=False,
        ),
        scratch_types=(
            [pltpu.VMEM_SHARED((N, dq), jnp.float32)]
            + [pltpu.VMEM_SHARED((N_PAD, dq), jnp.float32)]
            + [pltpu.VMEM((nsub, SUB), jnp.int32)] * 2
            + [pltpu.VMEM((nsub, SUB), jnp.int32)] * 2
            + [pltpu.VMEM((ch,), jnp.float32)] * 2
            + [pltpu.VMEM((SUB, dq), jnp.float32)] * 4
            + [pltpu.SemaphoreType.DMA] * 10
        ),
    )


def _blk(shape):
    return pl.BlockSpec(shape, lambda i: (i,) + (0,) * (len(shape) - 1))


def _full(shape):
    return pl.BlockSpec(shape, lambda i: (0,) * len(shape))


def _tc0_body(da, db, xr, dis_ref, u0_ref, u1_ref):
    deg = da[:, 0:1] + db[:, 0:1] + 1.0
    dis = jnp.where(deg > 0, lax.rsqrt(deg), 0.0)
    dis_ref[...] = dis
    u = dis * xr[...]
    u0_ref[...] = u[:, :64]
    u1_ref[...] = u[:, 64:]


def _tc0(deg_a, deg_b, x):
    return pl.pallas_call(
        _tc0_body,
        grid=(GRID,),
        in_specs=[_blk((BLK, D_OUT)), _blk((BLK, D_OUT)), _blk((BLK, D_IN))],
        out_specs=[_blk((BLK, 1)), _blk((BLK, 64)), _blk((BLK, 64))],
        out_shape=[jax.ShapeDtypeStruct((N, 1), jnp.float32),
                   jax.ShapeDtypeStruct((N, 64), jnp.float32),
                   jax.ShapeDtypeStruct((N, 64), jnp.float32)],
    )(deg_a, deg_b, x)


def _tc1_body(s0, s1, xr, dis, w1, b1, w2, *h_refs):
    d = dis[...]
    aggx = d * jnp.concatenate([s0[...], s1[...]], axis=1) + (d * d) * xr[...]
    z1 = jnp.dot(aggx, w1[...], preferred_element_type=jnp.float32) + b1[...]
    h2p = d * jnp.dot(z1, w2[...], preferred_element_type=jnp.float32)
    for q in range(4):
        h_refs[q][...] = h2p[:, 64 * q:64 * (q + 1)]


def _tc1(s10, s11, x, dis, w1, b1, w2):
    return pl.pallas_call(
        _tc1_body,
        grid=(GRID,),
        in_specs=[_blk((BLK, 64)), _blk((BLK, 64)), _blk((BLK, D_IN)),
                  _blk((BLK, 1)), _full((D_IN, D_H)), _full((1, D_H)),
                  _full((D_H, D_H))],
        out_specs=[_blk((BLK, 64))] * 4,
        out_shape=[jax.ShapeDtypeStruct((N, 64), jnp.float32)] * 4,
    )(s10, s11, x, dis, w1, b1, w2)


def _tc2_body(s0, s1, s2, s3, h0, h1, h2, h3, dis, b, w, *o_refs):
    d = dis[...]
    ss = (s0, s1, s2, s3)
    hh = (h0, h1, h2, h3)
    z = jnp.concatenate([d * (ss[q][...] + hh[q][...]) for q in range(4)],
                        axis=1) + b[...]
    hp = d * jnp.dot(z, w[...], preferred_element_type=jnp.float32)
    for q in range(4):
        o_refs[q][...] = hp[:, 64 * q:64 * (q + 1)]


def _tc2(sq, hq, dis, b, w):
    return pl.pallas_call(
        _tc2_body,
        grid=(GRID,),
        in_specs=[_blk((BLK, 64))] * 8
        + [_blk((BLK, 1)), _full((1, D_H)), _full((D_H, D_H))],
        out_specs=[_blk((BLK, 64))] * 4,
        out_shape=[jax.ShapeDtypeStruct((N, 64), jnp.float32)] * 4,
    )(*sq, *hq, dis, b, w)


def _tc3_body(s0, s1, s2, s3, h0, h1, h2, h3, dis, b, w, o_ref):
    d = dis[...]
    ss = (s0, s1, s2, s3)
    hh = (h0, h1, h2, h3)
    z = jnp.concatenate([d * (ss[q][...] + hh[q][...]) for q in range(4)],
                        axis=1) + b[...]
    h = jnp.maximum(z, 0.0)
    o_ref[...] = d * jnp.dot(h, w[...], preferred_element_type=jnp.float32)


def _tc3(sq, hq, dis, b, w):
    return pl.pallas_call(
        _tc3_body,
        grid=(GRID,),
        in_specs=[_blk((BLK, 64))] * 8
        + [_blk((BLK, 1)), _full((1, D_H)), _full((D_H, D_OUT))],
        out_specs=_blk((BLK, D_OUT)),
        out_shape=jax.ShapeDtypeStruct((N, D_OUT), jnp.float32),
    )(*sq, *hq, dis, b, w)


def _tc4_body(sa, sb, hp, dis, b, o_ref):
    d = dis[...]
    o_ref[...] = d * (sa[...] + sb[...] + hp[...]) + b[...]


def _tc4(sa, sb, hp, dis, b):
    return pl.pallas_call(
        _tc4_body,
        grid=(GRID,),
        in_specs=[_blk((BLK, D_OUT)), _blk((BLK, D_OUT)), _blk((BLK, D_OUT)),
                  _blk((BLK, 1)), _full((1, D_OUT))],
        out_specs=_blk((BLK, D_OUT)),
        out_shape=jax.ShapeDtypeStruct((N, D_OUT), jnp.float32),
    )(sa, sb, hp, dis, b)


def kernel(x, edge_index, edge_weight, W1, b1, W2, b2, W3, b3, W4, b4):
    src = edge_index[0]
    dst = edge_index[1]
    e = src.shape[0]
    pad = E_PAD - e
    pad_dst = (jnp.arange(pad, dtype=jnp.int32) * 97) % N
    src_r = jnp.concatenate([src, jnp.zeros((pad,), jnp.int32)]
                            ).reshape(E_PAD // SUB, SUB)
    dst_r = jnp.concatenate([dst, pad_dst]).reshape(E_PAD // SUB, SUB)
    ew_f = jnp.concatenate([edge_weight, jnp.zeros((pad,), jnp.float32)])

    ones16 = jnp.ones((N, D_OUT), jnp.float32)
    zeros16 = jnp.zeros((N_PAD, D_OUT), jnp.float32)
    zeros64 = jnp.zeros((N_PAD, 64), jnp.float32)

    agg16 = _build_agg(D_OUT, 1)
    agg2 = _build_agg(64, 2)
    agg4 = _build_agg(64, 4)

    def trim(arrs):
        return [a[:N] for a in arrs]

    deg_a, deg_b = trim(agg16(ones16, src_r, dst_r, ew_f, zeros16))
    dis, u0, u1 = _tc0(deg_a, deg_b, x)

    s10, s11 = trim(agg2(u0, u1, src_r, dst_r, ew_f, zeros64))
    h2q = _tc1(s10, s11, x, dis, W1, b1.reshape(1, -1), W2)

    s2q = trim(agg4(*h2q, src_r, dst_r, ew_f, zeros64))
    h3q = _tc2(s2q, h2q, dis, b2.reshape(1, -1), W3)

    s3q = trim(agg4(*h3q, src_r, dst_r, ew_f, zeros64))
    h4p = _tc3(s3q, h3q, dis, b3.reshape(1, -1), W4)

    s4a, s4b = trim(agg16(h4p, src_r, dst_r, ew_f, zeros16))
    out = _tc4(s4a, s4b, h4p, dis, b4.reshape(1, -1))
    return (out, 0)
